# baseline (speedup 1.0000x reference)
import functools

import jax
import jax.numpy as jnp
from jax import lax
from jax.experimental import pallas as pl
from jax.experimental.pallas import tpu as pltpu
from jax.experimental.pallas import tpu_sc as plsc

N_NODES = 10000
E_EDGES = 320000
N_GRAPHS = 128
N_CHUNKS = 32
CHUNK = E_EDGES // N_CHUNKS
LSUB = CHUNK // 16
GROWS = 79
GPAD = GROWS * 128

_mesh = plsc.VectorSubcoreMesh(core_axis_name="c", subcore_axis_name="s")

_LAST = None


def _bcast_last(v):
    idx = jnp.full((16,), 15, jnp.int32)
    return jnp.take_along_axis(v, idx, axis=0)


@functools.partial(
    pl.kernel,
    out_type=[
        jax.ShapeDtypeStruct((E_EDGES,), jnp.int32),
        jax.ShapeDtypeStruct((E_EDGES,), jnp.int32),
        jax.ShapeDtypeStruct((E_EDGES,), jnp.int32),
        jax.ShapeDtypeStruct((N_CHUNKS, N_GRAPHS), jnp.int32),
    ],
    mesh=_mesh,
    compiler_params=pltpu.CompilerParams(needs_layout_passes=False),
    scratch_types=[
        pltpu.VMEM((N_NODES,), jnp.int32),
        pltpu.VMEM((CHUNK,), jnp.int32),
        pltpu.VMEM((CHUNK,), jnp.int32),
        pltpu.VMEM((CHUNK,), jnp.int32),
        pltpu.VMEM((CHUNK,), jnp.int32),
        pltpu.VMEM((CHUNK,), jnp.int32),
        pltpu.VMEM((CHUNK,), jnp.int32),
        pltpu.VMEM((CHUNK,), jnp.int32),
        pltpu.VMEM((16, N_GRAPHS), jnp.int32),
        pltpu.VMEM((16, N_GRAPHS), jnp.int32),
        pltpu.VMEM((16, N_GRAPHS), jnp.int32),
        pltpu.VMEM((16, N_GRAPHS), jnp.int32),
        pltpu.VMEM((N_GRAPHS,), jnp.int32),
        pltpu.SemaphoreType.DMA,
    ],
)
def _stage_sorted(batch_hbm, ei_hbm, et_hbm,
                  stg_nin, stg_nout, stg_et, ct_out,
                  batch_v, nin_f, nout_f, et_f, snin, snout, set_, keys_f,
                  cnt_a, cnt_b, lcur_a, lcur_b, ct_v, sem):
    c = lax.axis_index("c")
    s = lax.axis_index("s")
    ka = c * 16 + s
    lane = lax.iota(jnp.int32, 16)
    l625 = lane * LSUB
    zeros16 = jnp.zeros((16,), jnp.int32)

    base_in = ka * CHUNK
    pltpu.sync_copy(batch_hbm, batch_v)
    pltpu.sync_copy(ei_hbm.at[pl.ds(base_in, CHUNK)], nin_f)
    pltpu.sync_copy(ei_hbm.at[pl.ds(E_EDGES + base_in, CHUNK)], nout_f)
    pltpu.sync_copy(et_hbm.at[pl.ds(base_in, CHUNK)], et_f)

    for l in range(16):
        for gc in range(N_GRAPHS // 16):
            cnt_a[l, pl.ds(gc * 16, 16)] = zeros16
            cnt_b[l, pl.ds(gc * 16, 16)] = zeros16

    HA = (LSUB + 1) // 2

    def count_half(t, cnt_ref, off):
        p = l625 + off + t
        nin_v = plsc.load_gather(nin_f, [p])
        g = plsc.load_gather(batch_v, [nin_v])
        keys_f[pl.ds((off + t) * 16, 16)] = g
        cvals = plsc.load_gather(cnt_ref, [lane, g])
        plsc.store_scatter(cnt_ref, [lane, g], cvals + 1)

    def count_body(t, carry):
        count_half(t, cnt_a, 0)
        count_half(t, cnt_b, HA)
        return carry
    lax.fori_loop(0, LSUB - HA, count_body, 0)
    count_half(LSUB - HA, cnt_a, 0)

    carry_l = jnp.int32(0)
    for gc in range(N_GRAPHS // 16):
        sl = pl.ds(gc * 16, 16)
        lbtot = zeros16
        for l in range(16):
            lbtot = lbtot + cnt_a[l, sl] + cnt_b[l, sl]
        ct_v[sl] = lbtot
        lb = plsc.cumsum(lbtot) - lbtot + carry_l
        carry_l = carry_l + jnp.sum(lbtot)
        runl = lb
        for l in range(16):
            lcur_a[l, sl] = runl
            lcur_b[l, sl] = runl + cnt_a[l, sl]
            runl = runl + cnt_a[l, sl] + cnt_b[l, sl]

    def place_half(t, lcur_ref, off):
        p = l625 + off + t
        g = keys_f[pl.ds((off + t) * 16, 16)]
        nin_v = plsc.load_gather(nin_f, [p])
        nout_v = plsc.load_gather(nout_f, [p])
        et_v = plsc.load_gather(et_f, [p])
        dl = plsc.load_gather(lcur_ref, [lane, g])
        plsc.store_scatter(lcur_ref, [lane, g], dl + 1)
        plsc.store_scatter(snin, [dl], nin_v)
        plsc.store_scatter(snout, [dl], nout_v)
        plsc.store_scatter(set_, [dl], et_v)

    def place_body(t, carry2):
        place_half(t, lcur_a, 0)
        place_half(t, lcur_b, HA)
        return carry2
    lax.fori_loop(0, LSUB - HA, place_body, 0)
    place_half(LSUB - HA, lcur_a, 0)

    base = ka * CHUNK
    c1 = pltpu.async_copy(snin, stg_nin.at[pl.ds(base, CHUNK)], sem)
    c2 = pltpu.async_copy(snout, stg_nout.at[pl.ds(base, CHUNK)], sem)
    c3 = pltpu.async_copy(set_, stg_et.at[pl.ds(base, CHUNK)], sem)
    c4 = pltpu.async_copy(ct_v, ct_out.at[ka], sem)
    c1.wait()
    c2.wait()
    c3.wait()
    c4.wait()


@functools.partial(
    pl.kernel,
    out_type=[
        jax.ShapeDtypeStruct((2 * E_EDGES,), jnp.int32),
        jax.ShapeDtypeStruct((E_EDGES,), jnp.int32),
    ],
    mesh=_mesh,
    compiler_params=pltpu.CompilerParams(needs_layout_passes=False),
    scratch_types=[
        pltpu.VMEM((N_CHUNKS, N_GRAPHS), jnp.int32),
        pltpu.VMEM((CHUNK,), jnp.int32),
        pltpu.VMEM((GROWS, 128), jnp.int32),
        pltpu.VMEM((GPAD,), jnp.int32),
        pltpu.VMEM((GPAD,), jnp.int32),
        pltpu.VMEM((GPAD,), jnp.int32),
        pltpu.SemaphoreType.DMA,
    ],
)
def _assemble(stg_nin, stg_nout, stg_et, ct_hbm, ei_out, et_out,
              ct_all, tdiff, perm2d, gnin, gnout, get_, sem):
    c = lax.axis_index("c")
    s = lax.axis_index("s")
    w = c * 16 + s
    qw = w * CHUNK
    qe = qw + CHUNK
    lane = lax.iota(jnp.int32, 16)
    zeros16 = jnp.zeros((16,), jnp.int32)

    pltpu.sync_copy(ct_hbm, ct_all)

    def zero_body(t, carry):
        for u in range(5):
            tdiff[pl.ds((t * 5 + u) * 16, 16)] = zeros16
        return carry
    lax.fori_loop(0, LSUB // 5, zero_body, 0)

    def run_body(g, carries):
        startrun, lb_lo, lb_hi, acc = carries
        gs = jnp.full((16,), g, jnp.int32)
        l_lo = plsc.load_gather(ct_all, [lane, gs])
        l_hi = plsc.load_gather(ct_all, [lane + 16, gs])
        cs_lo = plsc.cumsum(l_lo)
        sum_lo = _bcast_last(cs_lo)
        cs_hi = plsc.cumsum(l_hi) + sum_lo
        pre_lo = cs_lo - l_lo
        pre_hi = cs_hi - l_hi
        o_lo = startrun + pre_lo
        o_hi = startrun + pre_hi
        s_lo = lane * CHUNK + lb_lo
        s_hi = (lane + 16) * CHUNK + lb_hi
        for o, sv, lv in ((o_lo, s_lo, l_lo), (o_hi, s_hi, l_hi)):
            cc = sv - o
            nz = lv > 0
            bpos = o + lv
            in_a = jnp.logical_and(nz, jnp.logical_and(o > qw, o < qe))
            plsc.addupdate_scatter(tdiff, [o - qw], cc, mask=in_a)
            in_b = jnp.logical_and(nz, jnp.logical_and(bpos > qw, bpos < qe))
            plsc.addupdate_scatter(tdiff, [bpos - qw], -cc, mask=in_b)
            m0 = jnp.logical_and(nz, jnp.logical_and(o <= qw, bpos > qw))
            acc = acc + jnp.where(m0, cc, zeros16)
        startrun = startrun + _bcast_last(cs_hi)
        lb_lo = lb_lo + l_lo
        lb_hi = lb_hi + l_hi
        return startrun, lb_lo, lb_hi, acc
    _, _, _, acc = lax.fori_loop(
        0, N_GRAPHS, run_body, (zeros16, zeros16, zeros16, zeros16))
    c0 = jnp.sum(acc)

    def expand_body(t, carry):
        v = tdiff[pl.ds(t * 16, 16)]
        cs = plsc.cumsum(v) + carry
        perm2d[t // 8, pl.ds((t % 8) * 16, 16)] = cs + (qw + t * 16) + lane
        return _bcast_last(cs)
    lax.fori_loop(0, LSUB, expand_body, jnp.full((16,), c0, jnp.int32))
    for i in range(7):
        perm2d[GROWS - 1, pl.ds(16 + i * 16, 16)] = zeros16

    WIN = 16
    handles = []
    for j in range(GROWS):
        if j >= WIN:
            for cp in handles[3 * (j - WIN): 3 * (j - WIN) + 3]:
                cp.wait()
        handles.append(pltpu.async_copy(stg_nin.at[perm2d.at[j]],
                                        gnin.at[pl.ds(j * 128, 128)], sem))
        handles.append(pltpu.async_copy(stg_nout.at[perm2d.at[j]],
                                        gnout.at[pl.ds(j * 128, 128)], sem))
        handles.append(pltpu.async_copy(stg_et.at[perm2d.at[j]],
                                        get_.at[pl.ds(j * 128, 128)], sem))
    for cp in handles[3 * (GROWS - WIN):]:
        cp.wait()

    c1 = pltpu.async_copy(gnin.at[pl.ds(0, CHUNK)], ei_out.at[pl.ds(qw, CHUNK)], sem)
    c2 = pltpu.async_copy(gnout.at[pl.ds(0, CHUNK)], ei_out.at[pl.ds(E_EDGES + qw, CHUNK)], sem)
    c3 = pltpu.async_copy(get_.at[pl.ds(0, CHUNK)], et_out.at[pl.ds(qw, CHUNK)], sem)
    c1.wait()
    c2.wait()
    c3.wait()


def kernel(x, batch, edge_index, edge_type):
    stg_nin, stg_nout, stg_et, ct = _stage_sorted(
        batch.astype(jnp.int32), edge_index.astype(jnp.int32).reshape(2 * E_EDGES),
        edge_type.astype(jnp.int32))
    ei_flat, et_sorted = _assemble(stg_nin, stg_nout, stg_et, ct)
    edge_index_sorted = ei_flat.reshape(2, E_EDGES)
    edge_weight = jnp.ones((E_EDGES,), x.dtype)
    return x, edge_index_sorted, et_sorted, edge_weight


# speedup vs baseline: 1.0070x; 1.0070x over previous
import functools

import jax
import jax.numpy as jnp
from jax import lax
from jax.experimental import pallas as pl
from jax.experimental.pallas import tpu as pltpu
from jax.experimental.pallas import tpu_sc as plsc

N_NODES = 10000
E_EDGES = 320000
N_GRAPHS = 128
N_CHUNKS = 32
CHUNK = E_EDGES // N_CHUNKS
LSUB = CHUNK // 16
GROWS = 79
GPAD = GROWS * 128

_mesh = plsc.VectorSubcoreMesh(core_axis_name="c", subcore_axis_name="s")

_LAST = None


def _bcast_last(v):
    idx = jnp.full((16,), 15, jnp.int32)
    return jnp.take_along_axis(v, idx, axis=0)


@functools.partial(
    pl.kernel,
    out_type=[
        jax.ShapeDtypeStruct((E_EDGES,), jnp.int32),
        jax.ShapeDtypeStruct((E_EDGES,), jnp.int32),
        jax.ShapeDtypeStruct((E_EDGES,), jnp.int32),
        jax.ShapeDtypeStruct((N_CHUNKS, N_GRAPHS), jnp.int32),
    ],
    mesh=_mesh,
    compiler_params=pltpu.CompilerParams(needs_layout_passes=False),
    scratch_types=[
        pltpu.VMEM((N_NODES,), jnp.int32),
        pltpu.VMEM((CHUNK,), jnp.int32),
        pltpu.VMEM((CHUNK,), jnp.int32),
        pltpu.VMEM((CHUNK,), jnp.int32),
        pltpu.VMEM((CHUNK,), jnp.int32),
        pltpu.VMEM((CHUNK,), jnp.int32),
        pltpu.VMEM((CHUNK,), jnp.int32),
        pltpu.VMEM((16, N_GRAPHS), jnp.int32),
        pltpu.VMEM((16, N_GRAPHS), jnp.int32),
        pltpu.VMEM((N_GRAPHS,), jnp.int32),
        pltpu.SemaphoreType.DMA,
    ],
)
def _stage_sorted(batch_hbm, ei_hbm, et_hbm,
                  stg_nin, stg_nout, stg_et, ct_out,
                  batch_v, nin_f, nout_f, et_f, snin, snout, set_,
                  cnt, lcur, ct_v, sem):
    c = lax.axis_index("c")
    s = lax.axis_index("s")
    ka = c * 16 + s
    lane = lax.iota(jnp.int32, 16)
    l625 = lane * LSUB
    zeros16 = jnp.zeros((16,), jnp.int32)

    base_in = ka * CHUNK
    pltpu.sync_copy(batch_hbm, batch_v)
    pltpu.sync_copy(ei_hbm.at[pl.ds(base_in, CHUNK)], nin_f)
    ld1 = pltpu.async_copy(ei_hbm.at[pl.ds(E_EDGES + base_in, CHUNK)], nout_f, sem)
    ld2 = pltpu.async_copy(et_hbm.at[pl.ds(base_in, CHUNK)], et_f, sem)

    for l in range(16):
        for gc in range(N_GRAPHS // 16):
            cnt[l, pl.ds(gc * 16, 16)] = zeros16

    def count_body(t, carry):
        p = l625 + t
        nin_v = plsc.load_gather(nin_f, [p])
        g = plsc.load_gather(batch_v, [nin_v])
        cvals = plsc.load_gather(cnt, [lane, g])
        plsc.store_scatter(cnt, [lane, g], cvals + 1)
        return carry
    lax.fori_loop(0, LSUB, count_body, 0)

    carry_l = jnp.int32(0)
    for gc in range(N_GRAPHS // 16):
        sl = pl.ds(gc * 16, 16)
        lbtot = zeros16
        for l in range(16):
            lbtot = lbtot + cnt[l, sl]
        ct_v[sl] = lbtot
        lb = plsc.cumsum(lbtot) - lbtot + carry_l
        carry_l = carry_l + jnp.sum(lbtot)
        runl = lb
        for l in range(16):
            lcur[l, sl] = runl
            runl = runl + cnt[l, sl]

    ld1.wait()
    ld2.wait()

    def place_body(t, carry2):
        p = l625 + t
        nin_v = plsc.load_gather(nin_f, [p])
        g = plsc.load_gather(batch_v, [nin_v])
        nout_v = plsc.load_gather(nout_f, [p])
        et_v = plsc.load_gather(et_f, [p])
        dl = plsc.load_gather(lcur, [lane, g])
        plsc.store_scatter(lcur, [lane, g], dl + 1)
        plsc.store_scatter(snin, [dl], nin_v)
        plsc.store_scatter(snout, [dl], nout_v)
        plsc.store_scatter(set_, [dl], et_v)
        return carry2
    lax.fori_loop(0, LSUB, place_body, 0)

    base = ka * CHUNK
    c1 = pltpu.async_copy(snin, stg_nin.at[pl.ds(base, CHUNK)], sem)
    c2 = pltpu.async_copy(snout, stg_nout.at[pl.ds(base, CHUNK)], sem)
    c3 = pltpu.async_copy(set_, stg_et.at[pl.ds(base, CHUNK)], sem)
    c4 = pltpu.async_copy(ct_v, ct_out.at[ka], sem)
    c1.wait()
    c2.wait()
    c3.wait()
    c4.wait()


@functools.partial(
    pl.kernel,
    out_type=[
        jax.ShapeDtypeStruct((2 * E_EDGES,), jnp.int32),
        jax.ShapeDtypeStruct((E_EDGES,), jnp.int32),
    ],
    mesh=_mesh,
    compiler_params=pltpu.CompilerParams(needs_layout_passes=False),
    scratch_types=[
        pltpu.VMEM((N_CHUNKS, N_GRAPHS), jnp.int32),
        pltpu.VMEM((CHUNK,), jnp.int32),
        pltpu.VMEM((GROWS, 128), jnp.int32),
        pltpu.VMEM((GPAD,), jnp.int32),
        pltpu.VMEM((GPAD,), jnp.int32),
        pltpu.VMEM((GPAD,), jnp.int32),
        pltpu.SemaphoreType.DMA,
    ],
)
def _assemble(stg_nin, stg_nout, stg_et, ct_hbm, ei_out, et_out,
              ct_all, tdiff, perm2d, gnin, gnout, get_, sem):
    c = lax.axis_index("c")
    s = lax.axis_index("s")
    w = c * 16 + s
    qw = w * CHUNK
    qe = qw + CHUNK
    lane = lax.iota(jnp.int32, 16)
    zeros16 = jnp.zeros((16,), jnp.int32)

    ldct = pltpu.async_copy(ct_hbm, ct_all, sem)

    def zero_body(t, carry):
        for u in range(5):
            tdiff[pl.ds((t * 5 + u) * 16, 16)] = zeros16
        return carry
    lax.fori_loop(0, LSUB // 5, zero_body, 0)
    ldct.wait()

    def run_body(g, carries):
        startrun, lb_lo, lb_hi, acc = carries
        gs = jnp.full((16,), g, jnp.int32)
        l_lo = plsc.load_gather(ct_all, [lane, gs])
        l_hi = plsc.load_gather(ct_all, [lane + 16, gs])
        cs_lo = plsc.cumsum(l_lo)
        sum_lo = _bcast_last(cs_lo)
        cs_hi = plsc.cumsum(l_hi) + sum_lo
        pre_lo = cs_lo - l_lo
        pre_hi = cs_hi - l_hi
        o_lo = startrun + pre_lo
        o_hi = startrun + pre_hi
        s_lo = lane * CHUNK + lb_lo
        s_hi = (lane + 16) * CHUNK + lb_hi
        for o, sv, lv in ((o_lo, s_lo, l_lo), (o_hi, s_hi, l_hi)):
            cc = sv - o
            nz = lv > 0
            bpos = o + lv
            in_a = jnp.logical_and(nz, jnp.logical_and(o > qw, o < qe))
            plsc.addupdate_scatter(tdiff, [o - qw], cc, mask=in_a)
            in_b = jnp.logical_and(nz, jnp.logical_and(bpos > qw, bpos < qe))
            plsc.addupdate_scatter(tdiff, [bpos - qw], -cc, mask=in_b)
            m0 = jnp.logical_and(nz, jnp.logical_and(o <= qw, bpos > qw))
            acc = acc + jnp.where(m0, cc, zeros16)
        startrun = startrun + _bcast_last(cs_hi)
        lb_lo = lb_lo + l_lo
        lb_hi = lb_hi + l_hi
        return startrun, lb_lo, lb_hi, acc
    _, _, _, acc = lax.fori_loop(
        0, N_GRAPHS, run_body, (zeros16, zeros16, zeros16, zeros16))
    c0 = jnp.sum(acc)

    def expand_body(t, carry):
        v = tdiff[pl.ds(t * 16, 16)]
        cs = plsc.cumsum(v) + carry
        perm2d[t // 8, pl.ds((t % 8) * 16, 16)] = cs + (qw + t * 16) + lane
        return _bcast_last(cs)
    lax.fori_loop(0, LSUB, expand_body, jnp.full((16,), c0, jnp.int32))
    for i in range(7):
        perm2d[GROWS - 1, pl.ds(16 + i * 16, 16)] = zeros16

    WIN = 16
    handles = []
    for j in range(GROWS):
        if j >= WIN:
            for cp in handles[3 * (j - WIN): 3 * (j - WIN) + 3]:
                cp.wait()
        handles.append(pltpu.async_copy(stg_nin.at[perm2d.at[j]],
                                        gnin.at[pl.ds(j * 128, 128)], sem))
        handles.append(pltpu.async_copy(stg_nout.at[perm2d.at[j]],
                                        gnout.at[pl.ds(j * 128, 128)], sem))
        handles.append(pltpu.async_copy(stg_et.at[perm2d.at[j]],
                                        get_.at[pl.ds(j * 128, 128)], sem))
    for cp in handles[3 * (GROWS - WIN):]:
        cp.wait()

    c1 = pltpu.async_copy(gnin.at[pl.ds(0, CHUNK)], ei_out.at[pl.ds(qw, CHUNK)], sem)
    c2 = pltpu.async_copy(gnout.at[pl.ds(0, CHUNK)], ei_out.at[pl.ds(E_EDGES + qw, CHUNK)], sem)
    c3 = pltpu.async_copy(get_.at[pl.ds(0, CHUNK)], et_out.at[pl.ds(qw, CHUNK)], sem)
    c1.wait()
    c2.wait()
    c3.wait()


def kernel(x, batch, edge_index, edge_type):
    stg_nin, stg_nout, stg_et, ct = _stage_sorted(
        batch.astype(jnp.int32), edge_index.astype(jnp.int32).reshape(2 * E_EDGES),
        edge_type.astype(jnp.int32))
    ei_flat, et_sorted = _assemble(stg_nin, stg_nout, stg_et, ct)
    edge_index_sorted = ei_flat.reshape(2, E_EDGES)
    edge_weight = jnp.ones((E_EDGES,), x.dtype)
    return x, edge_index_sorted, et_sorted, edge_weight


# speedup vs baseline: 1.0090x; 1.0020x over previous
import functools

import jax
import jax.numpy as jnp
from jax import lax
from jax.experimental import pallas as pl
from jax.experimental.pallas import tpu as pltpu
from jax.experimental.pallas import tpu_sc as plsc

N_NODES = 10000
E_EDGES = 320000
N_GRAPHS = 128
N_CHUNKS = 32
CHUNK = E_EDGES // N_CHUNKS
LSUB = CHUNK // 16
GROWS = 79
GPAD = GROWS * 128

_mesh = plsc.VectorSubcoreMesh(core_axis_name="c", subcore_axis_name="s")

_LAST = None


def _bcast_last(v):
    idx = jnp.full((16,), 15, jnp.int32)
    return jnp.take_along_axis(v, idx, axis=0)


@functools.partial(
    pl.kernel,
    out_type=[
        jax.ShapeDtypeStruct((E_EDGES,), jnp.int32),
        jax.ShapeDtypeStruct((E_EDGES,), jnp.int32),
        jax.ShapeDtypeStruct((E_EDGES,), jnp.int32),
        jax.ShapeDtypeStruct((N_CHUNKS, N_GRAPHS), jnp.int32),
    ],
    mesh=_mesh,
    compiler_params=pltpu.CompilerParams(needs_layout_passes=False),
    scratch_types=[
        pltpu.VMEM((N_NODES,), jnp.int32),
        pltpu.VMEM((CHUNK,), jnp.int32),
        pltpu.VMEM((CHUNK,), jnp.int32),
        pltpu.VMEM((CHUNK,), jnp.int32),
        pltpu.VMEM((CHUNK,), jnp.int32),
        pltpu.VMEM((CHUNK,), jnp.int32),
        pltpu.VMEM((CHUNK,), jnp.int32),
        pltpu.VMEM((16, N_GRAPHS), jnp.int32),
        pltpu.VMEM((16, N_GRAPHS), jnp.int32),
        pltpu.VMEM((N_GRAPHS,), jnp.int32),
        pltpu.SemaphoreType.DMA,
    ],
)
def _stage_sorted(batch_hbm, ei_hbm, et_hbm,
                  stg_nin, stg_nout, stg_et, ct_out,
                  batch_v, nin_f, nout_f, et_f, snin, snout, set_,
                  cnt, lcur, ct_v, sem):
    c = lax.axis_index("c")
    s = lax.axis_index("s")
    ka = c * 16 + s
    lane = lax.iota(jnp.int32, 16)
    l625 = lane * LSUB
    zeros16 = jnp.zeros((16,), jnp.int32)

    base_in = ka * CHUNK
    pltpu.sync_copy(batch_hbm, batch_v)
    pltpu.sync_copy(ei_hbm.at[pl.ds(base_in, CHUNK)], nin_f)
    ld1 = pltpu.async_copy(ei_hbm.at[pl.ds(E_EDGES + base_in, CHUNK)], nout_f, sem)
    ld2 = pltpu.async_copy(et_hbm.at[pl.ds(base_in, CHUNK)], et_f, sem)

    for l in range(16):
        for gc in range(N_GRAPHS // 16):
            cnt[l, pl.ds(gc * 16, 16)] = zeros16

    def count_body(t, carry):
        p = l625 + t
        nin_v = plsc.load_gather(nin_f, [p])
        g = plsc.load_gather(batch_v, [nin_v])
        cvals = plsc.load_gather(cnt, [lane, g])
        plsc.store_scatter(cnt, [lane, g], cvals + 1)
        return carry
    lax.fori_loop(0, LSUB, count_body, 0)

    carry_l = jnp.int32(0)
    for gc in range(N_GRAPHS // 16):
        sl = pl.ds(gc * 16, 16)
        lbtot = zeros16
        for l in range(16):
            lbtot = lbtot + cnt[l, sl]
        ct_v[sl] = lbtot
        lb = plsc.cumsum(lbtot) - lbtot + carry_l
        carry_l = carry_l + jnp.sum(lbtot)
        runl = lb
        for l in range(16):
            lcur[l, sl] = runl
            runl = runl + cnt[l, sl]

    ld1.wait()
    ld2.wait()

    def place_body(t, carry2):
        p = l625 + t
        nin_v = plsc.load_gather(nin_f, [p])
        g = plsc.load_gather(batch_v, [nin_v])
        nout_v = plsc.load_gather(nout_f, [p])
        et_v = plsc.load_gather(et_f, [p])
        dl = plsc.load_gather(lcur, [lane, g])
        plsc.store_scatter(lcur, [lane, g], dl + 1)
        plsc.store_scatter(snin, [dl], nin_v)
        plsc.store_scatter(snout, [dl], nout_v)
        plsc.store_scatter(set_, [dl], et_v)
        return carry2
    lax.fori_loop(0, LSUB, place_body, 0)

    base = ka * CHUNK
    c1 = pltpu.async_copy(snin, stg_nin.at[pl.ds(base, CHUNK)], sem)
    c2 = pltpu.async_copy(snout, stg_nout.at[pl.ds(base, CHUNK)], sem)
    c3 = pltpu.async_copy(set_, stg_et.at[pl.ds(base, CHUNK)], sem)
    c4 = pltpu.async_copy(ct_v, ct_out.at[ka], sem)
    c1.wait()
    c2.wait()
    c3.wait()
    c4.wait()


@functools.partial(
    pl.kernel,
    out_type=[
        jax.ShapeDtypeStruct((2 * E_EDGES,), jnp.int32),
        jax.ShapeDtypeStruct((E_EDGES,), jnp.int32),
    ],
    mesh=_mesh,
    compiler_params=pltpu.CompilerParams(needs_layout_passes=False),
    scratch_types=[
        pltpu.VMEM((N_CHUNKS, N_GRAPHS), jnp.int32),
        pltpu.VMEM((CHUNK,), jnp.int32),
        pltpu.VMEM((CHUNK,), jnp.int32),
        pltpu.VMEM((LSUB + 15, ), jnp.int32),
        pltpu.VMEM((GROWS, 128), jnp.int32),
        pltpu.VMEM((GPAD,), jnp.int32),
        pltpu.VMEM((GPAD,), jnp.int32),
        pltpu.VMEM((GPAD,), jnp.int32),
        pltpu.SemaphoreType.DMA,
    ],
)
def _assemble(stg_nin, stg_nout, stg_et, ct_hbm, ei_out, et_out,
              ct_all, tdiff, csum, bpref, perm2d, gnin, gnout, get_, sem):
    c = lax.axis_index("c")
    s = lax.axis_index("s")
    w = c * 16 + s
    qw = w * CHUNK
    qe = qw + CHUNK
    lane = lax.iota(jnp.int32, 16)
    zeros16 = jnp.zeros((16,), jnp.int32)

    ldct = pltpu.async_copy(ct_hbm, ct_all, sem)

    def zero_body(t, carry):
        for u in range(5):
            tdiff[pl.ds((t * 5 + u) * 16, 16)] = zeros16
        return carry
    lax.fori_loop(0, LSUB // 5, zero_body, 0)
    ldct.wait()

    def run_body(g, carries):
        startrun, lb_lo, lb_hi, acc = carries
        gs = jnp.full((16,), g, jnp.int32)
        l_lo = plsc.load_gather(ct_all, [lane, gs])
        l_hi = plsc.load_gather(ct_all, [lane + 16, gs])
        cs_lo = plsc.cumsum(l_lo)
        sum_lo = _bcast_last(cs_lo)
        cs_hi = plsc.cumsum(l_hi) + sum_lo
        pre_lo = cs_lo - l_lo
        pre_hi = cs_hi - l_hi
        o_lo = startrun + pre_lo
        o_hi = startrun + pre_hi
        s_lo = lane * CHUNK + lb_lo
        s_hi = (lane + 16) * CHUNK + lb_hi
        for o, sv, lv in ((o_lo, s_lo, l_lo), (o_hi, s_hi, l_hi)):
            cc = sv - o
            nz = lv > 0
            bpos = o + lv
            in_a = jnp.logical_and(nz, jnp.logical_and(o > qw, o < qe))
            plsc.addupdate_scatter(tdiff, [o - qw], cc, mask=in_a)
            in_b = jnp.logical_and(nz, jnp.logical_and(bpos > qw, bpos < qe))
            plsc.addupdate_scatter(tdiff, [bpos - qw], -cc, mask=in_b)
            m0 = jnp.logical_and(nz, jnp.logical_and(o <= qw, bpos > qw))
            acc = acc + jnp.where(m0, cc, zeros16)
        startrun = startrun + _bcast_last(cs_hi)
        lb_lo = lb_lo + l_lo
        lb_hi = lb_hi + l_hi
        return startrun, lb_lo, lb_hi, acc
    _, _, _, acc = lax.fori_loop(
        0, N_GRAPHS, run_body, (zeros16, zeros16, zeros16, zeros16))
    c0 = jnp.sum(acc)

    @plsc.parallel_loop(0, LSUB, step=1)
    def _rowscan(t):
        v = tdiff[pl.ds(t * 16, 16)]
        csum[pl.ds(t * 16, 16)] = plsc.cumsum(v)

    def bpref_body(tb, carry):
        idx = (tb * 16 + lane) * 16 + 15
        s16 = plsc.load_gather(csum, [idx])
        cs2 = plsc.cumsum(s16) + carry
        bpref[pl.ds(tb * 16, 16)] = cs2 - s16
        return _bcast_last(cs2)
    lax.fori_loop(0, (LSUB + 15) // 16, bpref_body, jnp.full((16,), c0, jnp.int32))

    @plsc.parallel_loop(0, LSUB, step=1)
    def _combine(t):
        b = plsc.load_gather(bpref, [jnp.full((16,), t, jnp.int32)])
        v = csum[pl.ds(t * 16, 16)]
        perm2d[t // 8, pl.ds((t % 8) * 16, 16)] = v + b + (qw + t * 16) + lane
    for i in range(7):
        perm2d[GROWS - 1, pl.ds(16 + i * 16, 16)] = zeros16

    WIN = 16
    handles = []
    for j in range(GROWS):
        if j >= WIN:
            for cp in handles[3 * (j - WIN): 3 * (j - WIN) + 3]:
                cp.wait()
        handles.append(pltpu.async_copy(stg_nin.at[perm2d.at[j]],
                                        gnin.at[pl.ds(j * 128, 128)], sem))
        handles.append(pltpu.async_copy(stg_nout.at[perm2d.at[j]],
                                        gnout.at[pl.ds(j * 128, 128)], sem))
        handles.append(pltpu.async_copy(stg_et.at[perm2d.at[j]],
                                        get_.at[pl.ds(j * 128, 128)], sem))
    for cp in handles[3 * (GROWS - WIN):]:
        cp.wait()

    c1 = pltpu.async_copy(gnin.at[pl.ds(0, CHUNK)], ei_out.at[pl.ds(qw, CHUNK)], sem)
    c2 = pltpu.async_copy(gnout.at[pl.ds(0, CHUNK)], ei_out.at[pl.ds(E_EDGES + qw, CHUNK)], sem)
    c3 = pltpu.async_copy(get_.at[pl.ds(0, CHUNK)], et_out.at[pl.ds(qw, CHUNK)], sem)
    c1.wait()
    c2.wait()
    c3.wait()


def kernel(x, batch, edge_index, edge_type):
    stg_nin, stg_nout, stg_et, ct = _stage_sorted(
        batch.astype(jnp.int32), edge_index.astype(jnp.int32).reshape(2 * E_EDGES),
        edge_type.astype(jnp.int32))
    ei_flat, et_sorted = _assemble(stg_nin, stg_nout, stg_et, ct)
    edge_index_sorted = ei_flat.reshape(2, E_EDGES)
    edge_weight = jnp.ones((E_EDGES,), x.dtype)
    return x, edge_index_sorted, et_sorted, edge_weight
